# baseline (speedup 1.0000x reference)
import functools

import jax
import jax.numpy as jnp
from jax import lax
from jax.experimental import pallas as pl
from jax.experimental.pallas import tpu as pltpu
from jax.experimental.pallas import tpu_sc as plsc

B = 16384
NUM_BINS = 10
D = 64
EPS = 1e-5

NC = 2
NS = 16
NW = NC * NS
BPW = B // NW


IC = 128
NCHUNK = BPW // IC


@functools.lru_cache(maxsize=None)
def _get_sc_gather():
    mesh = plsc.VectorSubcoreMesh(core_axis_name="c", subcore_axis_name="s",
                                  num_cores=NC, num_subcores=NS)

    @functools.partial(
        pl.kernel,
        mesh=mesh,
        out_type=jax.ShapeDtypeStruct((4, B, D), jnp.float32),
        scratch_types=[
            pltpu.VMEM((NCHUNK, IC), jnp.int32),
            pltpu.VMEM((NCHUNK, IC), jnp.int32),
            pltpu.VMEM((BPW, D), jnp.float32),
            pltpu.VMEM((BPW, D), jnp.float32),
            pltpu.SemaphoreType.DMA,
            pltpu.SemaphoreType.DMA,
        ],
        compiler_params=pltpu.CompilerParams(use_tc_tiling_on_sc=False),
    )
    def _sc_gather(e0, e1, e2, e3, i0, i1, i2, i3, out, idx_a, idx_b, rows_a, rows_b, sem_a, sem_b):
        wid = lax.axis_index("s") * NC + lax.axis_index("c")
        cbase = wid * NCHUNK
        base = wid * BPW
        tables = (e0, e1, e2, e3)
        idxs = (i0, i1, i2, i3)
        ibufs = (idx_a, idx_b)
        bufs = (rows_a, rows_b)
        sems = (sem_a, sem_b)

        def start(f):
            ib, buf, sem = ibufs[f % 2], bufs[f % 2], sems[f % 2]
            pltpu.sync_copy(idxs[f].at[pl.ds(cbase, NCHUNK)], ib)
            for j in range(NCHUNK):
                pltpu.async_copy(tables[f].at[ib.at[j]], buf.at[pl.ds(j * IC, IC)], sem)
            return ib, buf, sem

        def drain(f, ib, buf, sem):
            for j in range(NCHUNK):
                pltpu.make_async_copy(tables[f].at[ib.at[j]], buf.at[pl.ds(j * IC, IC)], sem).wait()
            pltpu.sync_copy(buf, out.at[f, pl.ds(base, BPW)])

        cur = start(0)
        for f in range(4):
            if f < 3:
                nxt = start(f + 1)
            drain(f, *cur)
            if f < 3:
                cur = nxt

    return _sc_gather


def _layernorm(tok, gamma, beta):
    mu = jnp.mean(tok, axis=-1, keepdims=True)
    xc = tok - mu
    var = jnp.mean(xc * xc, axis=-1, keepdims=True)
    return xc * lax.rsqrt(var + EPS) * gamma + beta


def _tc_body(nums_ref, g_ref, centers_ref, wt_ref, bias_ref, gamma_ref, beta_ref, out_ref):
    gamma = gamma_ref[0:1, :]
    beta = beta_ref[0:1, :]
    for f in range(4):
        x = nums_ref[:, f:f + 1]
        mask = jnp.isnan(x)
        clean = jnp.where(mask, 0.0, x)
        d = -((clean - centers_ref[f:f + 1, :]) ** 2)
        d = d - jnp.max(d, axis=-1, keepdims=True)
        e = jnp.exp(d)
        p = e / jnp.sum(e, axis=-1, keepdims=True)
        tok = jnp.dot(p, wt_ref[f], preferred_element_type=jnp.float32)
        tok = tok + bias_ref[f:f + 1, :]
        tok = jnp.where(mask, 0.0, tok)
        out_ref[:, f * D:(f + 1) * D] = _layernorm(tok, gamma, beta)
    for f in range(4):
        tok = g_ref[f]
        out_ref[:, (4 + f) * D:(5 + f) * D] = _layernorm(tok, gamma, beta)


BM = 2048


def _tc_call(nums, g, centers, wt, bias, gamma, beta, interpret=False):
    grid = B // BM
    return pl.pallas_call(
        _tc_body,
        grid=(grid,),
        in_specs=[
            pl.BlockSpec((BM, 4), lambda i: (i, 0)),
            pl.BlockSpec((4, BM, D), lambda i: (0, i, 0)),
            pl.BlockSpec((4, NUM_BINS), lambda i: (0, 0)),
            pl.BlockSpec((4, NUM_BINS, D), lambda i: (0, 0, 0)),
            pl.BlockSpec((4, D), lambda i: (0, 0)),
            pl.BlockSpec((1, D), lambda i: (0, 0)),
            pl.BlockSpec((1, D), lambda i: (0, 0)),
        ],
        out_specs=pl.BlockSpec((BM, 8 * D), lambda i: (i, 0)),
        out_shape=jax.ShapeDtypeStruct((B, 8 * D), jnp.float32),
        interpret=interpret,
    )(nums, g, centers, wt, bias, gamma, beta)


@jax.jit
def kernel(num_0, num_1, num_2, num_3, cat_0, cat_1, cat_2, cat_3,
           centers_0, centers_1, centers_2, centers_3,
           W_0, W_1, W_2, W_3, b_0, b_1, b_2, b_3,
           E_0, E_1, E_2, E_3, gamma, beta):
    cats = [c.reshape(B // IC, IC) for c in (cat_0, cat_1, cat_2, cat_3)]
    g = _get_sc_gather()(E_0, E_1, E_2, E_3, *cats)
    nums = jnp.stack([num_0, num_1, num_2, num_3], axis=1)
    centers = jnp.stack([centers_0, centers_1, centers_2, centers_3])
    wt = jnp.stack([W_0.T, W_1.T, W_2.T, W_3.T])
    bias = jnp.stack([b_0, b_1, b_2, b_3])
    out = _tc_call(nums, g, centers, wt, bias, gamma[None, :], beta[None, :])
    return out.reshape(B, 8, D)
